# baseline (speedup 1.0000x reference)
import functools

import jax
import jax.numpy as jnp
from jax import lax
from jax.experimental import pallas as pl
from jax.experimental.pallas import tpu as pltpu
from jax.experimental.pallas import tpu_sc as plsc

VOCAB = 100000
D_MODEL = 1024
BATCH = 4
SEQ = 2048

_INFO = plsc.get_sparse_core_info()
_NC = _INFO.num_cores
_NS = _INFO.num_subcores
_NW = _NC * _NS
_SPW = SEQ // _NW
_C = 32
_NH = _SPW // _C
_NCHUNK = _NH * BATCH
_NBUF = 3
_L = 16
_PSL = D_MODEL // (2 * _L)


def _emb_body(x_hbm, tab_hbm, pe_hbm, out_hbm,
              idx_v, buf0, buf1, buf2, pbuf,
              g0, g1, g2, s0sem, s1sem, s2sem, isem, psem):
    wid = lax.axis_index("s") * _NC + lax.axis_index("c")
    s0 = wid * _SPW
    bufs = (buf0, buf1, buf2)
    gsems = (g0, g1, g2)
    ssems = (s0sem, s1sem, s2sem)

    pltpu.sync_copy(x_hbm.at[pl.ds(wid * BATCH * _SPW, BATCH * _SPW)], idx_v)

    def chunk_hb(i):
        return i // BATCH, i % BATCH

    def issue_gather(i):
        h, b = chunk_hb(i)
        return pltpu.async_copy(
            tab_hbm.at[idx_v.at[pl.ds(b * _SPW + h * _C, _C)]],
            bufs[i % _NBUF], gsems[i % _NBUF])

    def issue_store(i):
        h, b = chunk_hb(i)
        return pltpu.async_copy(
            bufs[i % _NBUF],
            out_hbm.at[pl.ds(b * SEQ + s0 + h * _C, _C)],
            ssems[i % _NBUF])

    gathers = [None] * _NCHUNK
    stores = [None] * _NCHUNK
    pe_cp = pltpu.async_copy(
        pe_hbm.at[pl.ds(s0 * (D_MODEL // 2), _C * (D_MODEL // 2))], pbuf, psem)
    for i in range(_NBUF - 1):
        gathers[i] = issue_gather(i)

    for i in range(_NCHUNK):
        nxt = i + _NBUF - 1
        if nxt < _NCHUNK:
            if nxt >= _NBUF:
                stores[nxt - _NBUF].wait()
            gathers[nxt] = issue_gather(nxt)
        gathers[i].wait()
        if i == 0:
            pe_cp.wait()
        h, _ = chunk_hb(i)
        if i == (_NCHUNK // _NH) and _NH > 1:
            pltpu.sync_copy(
                pe_hbm.at[pl.ds((s0 + h * _C) * (D_MODEL // 2),
                                _C * (D_MODEL // 2))], pbuf)
        buf = bufs[i % _NBUF]

        def row_add(r, _):
            pe_off = pl.multiple_of(r * (D_MODEL // 2), D_MODEL // 2)
            for k in range(_PSL):
                w = pbuf[pl.ds(pe_off + k * _L, _L)]
                a = lax.bitcast_convert_type(w << 16, jnp.float32)
                b2 = lax.bitcast_convert_type(
                    w & jnp.int32(-65536), jnp.float32)
                sla = pl.ds(k * 2 * _L, _L)
                slb = pl.ds(k * 2 * _L + _L, _L)
                buf[r, sla] = buf[r, sla] + a
                buf[r, slb] = buf[r, slb] + b2
            return ()

        lax.fori_loop(0, _C, row_add, ())
        stores[i] = issue_store(i)
    for i in range(_NCHUNK - _NBUF, _NCHUNK):
        stores[i].wait()


@jax.jit
def _emb(x_perm, tok_table, pe_sw):
    mesh = plsc.VectorSubcoreMesh(core_axis_name="c", subcore_axis_name="s")
    k = pl.kernel(
        _emb_body,
        out_type=jax.ShapeDtypeStruct((BATCH * SEQ, D_MODEL), jnp.float32),
        mesh=mesh,
        scratch_types=[
            pltpu.VMEM((BATCH * _SPW,), jnp.int32),
            pltpu.VMEM((_C, D_MODEL), jnp.float32),
            pltpu.VMEM((_C, D_MODEL), jnp.float32),
            pltpu.VMEM((_C, D_MODEL), jnp.float32),
            pltpu.VMEM((_C * (D_MODEL // 2),), jnp.int32),
            pltpu.SemaphoreType.DMA,
            pltpu.SemaphoreType.DMA,
            pltpu.SemaphoreType.DMA,
            pltpu.SemaphoreType.DMA,
            pltpu.SemaphoreType.DMA,
            pltpu.SemaphoreType.DMA,
            pltpu.SemaphoreType.DMA,
            pltpu.SemaphoreType.DMA,
        ],
    )
    return k(x_perm, tok_table, pe_sw)


def kernel(x, tok_table, pe):
    x_perm = x.reshape(BATCH, _NW, _SPW).transpose(1, 0, 2).reshape(-1)
    pe_bits = jax.lax.bitcast_convert_type(
        pe.astype(jnp.bfloat16), jnp.uint16).astype(jnp.uint32)
    blocks = pe_bits.reshape(SEQ, _PSL, 2, _L)
    words = (blocks[:, :, 1, :] << 16) | blocks[:, :, 0, :]
    pe_sw = jax.lax.bitcast_convert_type(
        words, jnp.int32).reshape(SEQ * (D_MODEL // 2))
    out = _emb(x_perm, tok_table, pe_sw)
    return out.reshape(BATCH, SEQ, D_MODEL)


# speedup vs baseline: 1.0070x; 1.0070x over previous
import functools

import jax
import jax.numpy as jnp
from jax import lax
from jax.experimental import pallas as pl
from jax.experimental.pallas import tpu as pltpu
from jax.experimental.pallas import tpu_sc as plsc

VOCAB = 100000
D_MODEL = 1024
BATCH = 4
SEQ = 2048

_INFO = plsc.get_sparse_core_info()
_NC = _INFO.num_cores
_NS = _INFO.num_subcores
_NW = _NC * _NS
_SPW = SEQ // _NW
_C = 32
_NH = _SPW // _C
_NCHUNK = _NH * BATCH
_NBUF = 2
_L = 16
_PSL = D_MODEL // (2 * _L)


def _emb_body(x_hbm, tab_hbm, pe_hbm, out_hbm,
              idx_v, buf0, buf1, pbuf,
              gsem, ssem, isem, psem):
    wid = lax.axis_index("s") * _NC + lax.axis_index("c")
    s0 = wid * _SPW
    bufs = (buf0, buf1)
    gsems = (gsem, gsem)
    ssems = (ssem, ssem)

    pltpu.sync_copy(x_hbm.at[pl.ds(wid * BATCH * _SPW, BATCH * _SPW)], idx_v)

    def chunk_hb(i):
        return i // BATCH, i % BATCH

    def issue_gather(i):
        h, b = chunk_hb(i)
        return pltpu.async_copy(
            tab_hbm.at[idx_v.at[pl.ds(b * _SPW + h * _C, _C)]],
            bufs[i % _NBUF], gsems[i % _NBUF])

    def issue_store(i):
        h, b = chunk_hb(i)
        return pltpu.async_copy(
            bufs[i % _NBUF],
            out_hbm.at[pl.ds(b * SEQ + s0 + h * _C, _C)],
            ssems[i % _NBUF])

    gathers = [None] * _NCHUNK
    stores = [None] * _NCHUNK
    pe_cp = pltpu.async_copy(
        pe_hbm.at[pl.ds(s0 * (D_MODEL // 2), _C * (D_MODEL // 2))], pbuf, psem)
    for i in range(_NBUF - 1):
        gathers[i] = issue_gather(i)

    for i in range(_NCHUNK):
        nxt = i + _NBUF - 1
        if nxt < _NCHUNK:
            if nxt >= _NBUF:
                stores[nxt - _NBUF].wait()
            gathers[nxt] = issue_gather(nxt)
        gathers[i].wait()
        if i == 0:
            pe_cp.wait()
        h, _ = chunk_hb(i)
        if i == (_NCHUNK // _NH) and _NH > 1:
            pltpu.sync_copy(
                pe_hbm.at[pl.ds((s0 + h * _C) * (D_MODEL // 2),
                                _C * (D_MODEL // 2))], pbuf)
        buf = bufs[i % _NBUF]

        def row_add(r, _):
            pe_off = pl.multiple_of(r * (D_MODEL // 2), D_MODEL // 2)
            for k in range(_PSL):
                w = pbuf[pl.ds(pe_off + k * _L, _L)]
                a = lax.bitcast_convert_type(w << 16, jnp.float32)
                b2 = lax.bitcast_convert_type(
                    w & jnp.int32(-65536), jnp.float32)
                sla = pl.ds(k * 2 * _L, _L)
                slb = pl.ds(k * 2 * _L + _L, _L)
                buf[r, sla] = buf[r, sla] + a
                buf[r, slb] = buf[r, slb] + b2
            return ()

        lax.fori_loop(0, _C, row_add, ())
        stores[i] = issue_store(i)
    for i in range(_NCHUNK - _NBUF, _NCHUNK):
        stores[i].wait()


@jax.jit
def _emb(x_perm, tok_table, pe_sw):
    mesh = plsc.VectorSubcoreMesh(core_axis_name="c", subcore_axis_name="s")
    k = pl.kernel(
        _emb_body,
        out_type=jax.ShapeDtypeStruct((BATCH * SEQ, D_MODEL), jnp.float32),
        mesh=mesh,
        scratch_types=[
            pltpu.VMEM((BATCH * _SPW,), jnp.int32),
            pltpu.VMEM((_C, D_MODEL), jnp.float32),
            pltpu.VMEM((_C, D_MODEL), jnp.float32),
            pltpu.VMEM((_C * (D_MODEL // 2),), jnp.int32),
            pltpu.SemaphoreType.DMA,
            pltpu.SemaphoreType.DMA,
            pltpu.SemaphoreType.DMA,
            pltpu.SemaphoreType.DMA,
        ],
    )
    return k(x_perm, tok_table, pe_sw)


def kernel(x, tok_table, pe):
    x_perm = x.reshape(BATCH, _NW, _SPW).transpose(1, 0, 2).reshape(-1)
    pe_bits = jax.lax.bitcast_convert_type(
        pe.astype(jnp.bfloat16), jnp.uint16).astype(jnp.uint32)
    blocks = pe_bits.reshape(SEQ, _PSL, 2, _L)
    words = (blocks[:, :, 1, :] << 16) | blocks[:, :, 0, :]
    pe_sw = jax.lax.bitcast_convert_type(
        words, jnp.int32).reshape(SEQ * (D_MODEL // 2))
    out = _emb(x_perm, tok_table, pe_sw)
    return out.reshape(BATCH, SEQ, D_MODEL)


# speedup vs baseline: 1.5013x; 1.4908x over previous
import functools

import jax
import jax.numpy as jnp
from jax import lax
from jax.experimental import pallas as pl
from jax.experimental.pallas import tpu as pltpu
from jax.experimental.pallas import tpu_sc as plsc

VOCAB = 100000
D_MODEL = 1024
BATCH = 4
SEQ = 2048

_INFO = plsc.get_sparse_core_info()
_NC = _INFO.num_cores
_NS = _INFO.num_subcores
_NW = _NC * _NS
_SPW = SEQ // _NW
_C = 32
_NH = _SPW // _C
_NCHUNK = _NH * BATCH
_NBUF = 2
_L = 16
_PSL = D_MODEL // (2 * _L)


def _emb_body(x_hbm, tab_hbm, pe_hbm, out_hbm,
              idx_v, buf0, buf1, pbuf,
              gsem, ssem, isem, psem):
    wid = lax.axis_index("s") * _NC + lax.axis_index("c")
    s0 = wid * _SPW
    bufs = (buf0, buf1)
    gsems = (gsem, gsem)
    ssems = (ssem, ssem)

    pltpu.sync_copy(x_hbm.at[pl.ds(wid * BATCH * _SPW, BATCH * _SPW)], idx_v)

    def chunk_hb(i):
        return i // BATCH, i % BATCH

    def issue_gather(i):
        h, b = chunk_hb(i)
        return pltpu.async_copy(
            tab_hbm.at[idx_v.at[pl.ds(b * _SPW + h * _C, _C)]],
            bufs[i % _NBUF], gsems[i % _NBUF])

    def issue_store(i):
        h, b = chunk_hb(i)
        return pltpu.async_copy(
            bufs[i % _NBUF],
            out_hbm.at[pl.ds(b * SEQ + s0 + h * _C, _C)],
            ssems[i % _NBUF])

    gathers = [None] * _NCHUNK
    stores = [None] * _NCHUNK
    pe_cp = pltpu.async_copy(
        pe_hbm.at[pl.ds(s0, _C)], pbuf, psem)
    for i in range(_NBUF - 1):
        gathers[i] = issue_gather(i)

    for i in range(_NCHUNK):
        nxt = i + _NBUF - 1
        if nxt < _NCHUNK:
            if nxt >= _NBUF:
                stores[nxt - _NBUF].wait()
            gathers[nxt] = issue_gather(nxt)
        gathers[i].wait()
        if i == 0:
            pe_cp.wait()
        h, _ = chunk_hb(i)
        if i == (_NCHUNK // _NH) and _NH > 1:
            pltpu.sync_copy(
                pe_hbm.at[pl.ds(s0 + h * _C, _C)], pbuf)
        buf = bufs[i % _NBUF]

        def row_add(r, _):
            for k in range(_PSL):
                w = pbuf[r, pl.ds(k * _L, _L)]
                a = lax.bitcast_convert_type(w << 16, jnp.float32)
                b2 = lax.bitcast_convert_type(
                    w & jnp.int32(-65536), jnp.float32)
                sla = pl.ds(k * 2 * _L, _L)
                slb = pl.ds(k * 2 * _L + _L, _L)
                buf[r, sla] = buf[r, sla] + a
                buf[r, slb] = buf[r, slb] + b2
            return ()

        lax.fori_loop(0, _C, row_add, ())
        stores[i] = issue_store(i)
    for i in range(_NCHUNK - _NBUF, _NCHUNK):
        stores[i].wait()


@jax.jit
def _emb(x_perm, tok_table, pe_sw):
    mesh = plsc.VectorSubcoreMesh(core_axis_name="c", subcore_axis_name="s")
    k = pl.kernel(
        _emb_body,
        out_type=jax.ShapeDtypeStruct((BATCH * SEQ, D_MODEL), jnp.float32),
        mesh=mesh,
        scratch_types=[
            pltpu.VMEM((BATCH * _SPW,), jnp.int32),
            pltpu.VMEM((_C, D_MODEL), jnp.float32),
            pltpu.VMEM((_C, D_MODEL), jnp.float32),
            pltpu.VMEM((_C, D_MODEL // 2), jnp.int32),
            pltpu.SemaphoreType.DMA,
            pltpu.SemaphoreType.DMA,
            pltpu.SemaphoreType.DMA,
            pltpu.SemaphoreType.DMA,
        ],
    )
    return k(x_perm, tok_table, pe_sw)


def kernel(x, tok_table, pe):
    x_perm = x.reshape(BATCH, _NW, _SPW).transpose(1, 0, 2).reshape(-1)
    pe_bits = jax.lax.bitcast_convert_type(
        pe.astype(jnp.bfloat16), jnp.uint16).astype(jnp.uint32)
    blocks = pe_bits.reshape(SEQ, _PSL, 2, _L)
    words = (blocks[:, :, 1, :] << 16) | blocks[:, :, 0, :]
    pe_sw = jax.lax.bitcast_convert_type(
        words, jnp.int32).reshape(SEQ, D_MODEL // 2)
    out = _emb(x_perm, tok_table, pe_sw)
    return out.reshape(BATCH, SEQ, D_MODEL)


# speedup vs baseline: 2.2831x; 1.5208x over previous
import functools

import jax
import jax.numpy as jnp
from jax import lax
from jax.experimental import pallas as pl
from jax.experimental.pallas import tpu as pltpu
from jax.experimental.pallas import tpu_sc as plsc

VOCAB = 100000
D_MODEL = 1024
BATCH = 4
SEQ = 2048

_INFO = plsc.get_sparse_core_info()
_NC = _INFO.num_cores
_NS = _INFO.num_subcores
_NW = _NC * _NS
_SPW = SEQ // _NW
_W = 24
_PH = (_W, _W, _SPW - 2 * _W)
_NPH = len(_PH)
_NCHUNK = _NPH * BATCH
_L = 16
_DSL = D_MODEL // _L


def _emb_body(x_hbm, tab_hbm, pe_hbm, out_hbm,
              idx_v, buf0, buf1, buf2, pbuf,
              g0, g1, g2, s0s, s1s, s2s, psem):
    wid = lax.axis_index("s") * _NC + lax.axis_index("c")
    s0 = wid * _SPW
    bufs = (buf0, buf1, buf2)
    gsems = (g0, g1, g2)
    ssems = (s0s, s1s, s2s)

    pltpu.sync_copy(x_hbm.at[pl.ds(wid * BATCH * _SPW, BATCH * _SPW)], idx_v)

    def chunk_pb(i):
        return i // BATCH, i % BATCH

    def issue_gather(i):
        p, b = chunk_pb(i)
        sz = _PH[p]
        return pltpu.async_copy(
            tab_hbm.at[idx_v.at[pl.ds(b * _SPW + p * _W, sz)]],
            bufs[i % 3].at[pl.ds(0, sz)], gsems[i % 3])

    def issue_store(i):
        p, b = chunk_pb(i)
        sz = _PH[p]
        return pltpu.async_copy(
            bufs[i % 3].at[pl.ds(0, sz)],
            out_hbm.at[pl.ds(b * SEQ + s0 + p * _W, sz)],
            ssems[i % 3])

    gathers = [None] * _NCHUNK
    stores = [None] * _NCHUNK
    pe_cp = pltpu.async_copy(pe_hbm.at[pl.ds(s0, _PH[0])], pbuf, psem)
    gathers[0] = issue_gather(0)

    for i in range(_NCHUNK):
        if i + 1 < _NCHUNK:
            if i >= 2:
                stores[i - 2].wait()
            gathers[i + 1] = issue_gather(i + 1)
        gathers[i].wait()
        p, _b = chunk_pb(i)
        if i == 0:
            pe_cp.wait()
        elif i % BATCH == 0:
            pltpu.sync_copy(pe_hbm.at[pl.ds(s0 + p * _W, _PH[p])],
                            pbuf.at[pl.ds(0, _PH[p])])
        buf = bufs[i % 3]

        def row_add(r, _):
            for k in range(_DSL):
                sl = pl.ds(k * _L, _L)
                buf[r, sl] = buf[r, sl] + pbuf[r, sl]
            return ()

        lax.fori_loop(0, _PH[p], row_add, ())
        stores[i] = issue_store(i)
    stores[_NCHUNK - 2].wait()
    stores[_NCHUNK - 1].wait()


@jax.jit
def _emb(x_perm, tok_table, pe):
    mesh = plsc.VectorSubcoreMesh(core_axis_name="c", subcore_axis_name="s")
    k = pl.kernel(
        _emb_body,
        out_type=jax.ShapeDtypeStruct((BATCH * SEQ, D_MODEL), jnp.float32),
        mesh=mesh,
        scratch_types=[
            pltpu.VMEM((BATCH * _SPW,), jnp.int32),
            pltpu.VMEM((_W, D_MODEL), jnp.float32),
            pltpu.VMEM((_W, D_MODEL), jnp.float32),
            pltpu.VMEM((_W, D_MODEL), jnp.float32),
            pltpu.VMEM((_W, D_MODEL), jnp.float32),
            pltpu.SemaphoreType.DMA,
            pltpu.SemaphoreType.DMA,
            pltpu.SemaphoreType.DMA,
            pltpu.SemaphoreType.DMA,
            pltpu.SemaphoreType.DMA,
            pltpu.SemaphoreType.DMA,
            pltpu.SemaphoreType.DMA,
        ],
    )
    return k(x_perm, tok_table, pe)


def kernel(x, tok_table, pe):
    x_perm = x.reshape(BATCH, _NW, _SPW).transpose(1, 0, 2).reshape(-1)
    out = _emb(x_perm, tok_table, pe)
    return out.reshape(BATCH, SEQ, D_MODEL)
